# Initial kernel scaffold; baseline (speedup 1.0000x reference)
import jax
import jax.numpy as jnp
from jax.experimental import pallas as pl


def kernel(onehot, idx_to_name):
    raise NotImplementedError("write your pallas kernel here")


# speedup vs baseline: 5.5762x; 5.5762x over previous
import functools

import jax
import jax.numpy as jnp
from jax import lax
from jax.experimental import pallas as pl
from jax.experimental.pallas import tpu as pltpu
from jax.experimental.pallas import tpu_sc as plsc

BATCH = 4096
SEQ = 50
NUM_CLASSES = 100
ROWS = BATCH * SEQ
NUM_CORES = 2
NUM_SUBCORES = 16
NW = NUM_CORES * NUM_SUBCORES
ROWS_PER_W = ROWS // NW
CHUNK_ROWS = 400
NUM_CHUNKS = ROWS_PER_W // CHUNK_ROWS
GROUPS = CHUNK_ROWS // 16
TABLE_PAD = 128

_mesh = plsc.VectorSubcoreMesh(core_axis_name="c", subcore_axis_name="s")


@functools.partial(
    pl.kernel,
    out_type=jax.ShapeDtypeStruct((ROWS,), jnp.int32),
    mesh=_mesh,
    scratch_types=[
        pltpu.VMEM((CHUNK_ROWS * NUM_CLASSES,), jnp.float32),
        pltpu.VMEM((CHUNK_ROWS,), jnp.int32),
        pltpu.VMEM((TABLE_PAD,), jnp.int32),
    ],
    compiler_params=pltpu.CompilerParams(needs_layout_passes=False),
)
def _sc_onehot_to_name(onehot_hbm, table_hbm, out_hbm, inbuf, outbuf, table_v):
    wid = lax.axis_index("s") * NUM_CORES + lax.axis_index("c")
    row0 = wid * ROWS_PER_W
    pltpu.sync_copy(table_hbm, table_v)
    lane_base = lax.iota(jnp.int32, 16) * NUM_CLASSES

    def chunk_body(ch, carry):
        base = (row0 + ch * CHUNK_ROWS) * NUM_CLASSES
        pltpu.sync_copy(onehot_hbm.at[pl.ds(base, CHUNK_ROWS * NUM_CLASSES)], inbuf)

        def group_body(g, carry2):
            idx0 = lane_base + g * (16 * NUM_CLASSES)
            accs = [jnp.zeros((16,), jnp.float32) for _ in range(4)]
            for c in range(1, NUM_CLASSES):
                v = plsc.load_gather(inbuf, [idx0 + c])
                accs[c % 4] = accs[c % 4] + v * float(c)
            acc = (accs[0] + accs[1]) + (accs[2] + accs[3])
            names = plsc.load_gather(table_v, [acc.astype(jnp.int32)])
            outbuf[pl.ds(g * 16, 16)] = names
            return carry2

        lax.fori_loop(0, GROUPS, group_body, 0)
        pltpu.sync_copy(outbuf, out_hbm.at[pl.ds(row0 + ch * CHUNK_ROWS, CHUNK_ROWS)])
        return carry

    lax.fori_loop(0, NUM_CHUNKS, chunk_body, 0)


def kernel(onehot, idx_to_name):
    flat = onehot.reshape(ROWS * NUM_CLASSES)
    table = jnp.zeros((TABLE_PAD,), jnp.int32).at[:NUM_CLASSES].set(idx_to_name)
    out = _sc_onehot_to_name(flat, table)
    return out.reshape(BATCH, SEQ)
